# baseline (speedup 1.0000x reference)
import jax
import jax.numpy as jnp
from jax.experimental import pallas as pl


def _layernorm(x, gamma, beta, eps=1e-5):
    mu = jnp.mean(x, axis=-1, keepdims=True)
    var = jnp.var(x, axis=-1, keepdims=True)
    return (x - mu) / jnp.sqrt(var + eps) * gamma + beta


def _gat_conv(x, src, dst, ea, lin, a_src, a_dst, lin_e, a_e, bias, n):
    loops = jnp.arange(n, dtype=src.dtype)
    src2 = jnp.concatenate([src, loops])
    dst2 = jnp.concatenate([dst, loops])
    mean_ea = jnp.mean(ea, axis=0, keepdims=True)
    ea2 = jnp.concatenate([ea, jnp.broadcast_to(mean_ea, (n, ea.shape[1]))], axis=0)
    h = x @ lin
    al_src = (h * a_src).sum(-1)
    al_dst = (h * a_dst).sum(-1)
    he = ea2 @ lin_e
    al_e = (he * a_e).sum(-1)
    alpha = jax.nn.leaky_relu(al_src[src2] + al_dst[dst2] + al_e, negative_slope=0.2)
    amax = jax.ops.segment_max(alpha, dst2, num_segments=n)
    alpha = jnp.exp(alpha - amax[dst2])
    denom = jax.ops.segment_sum(alpha, dst2, num_segments=n)
    alpha = alpha / (denom[dst2] + 1e-16)
    out = jax.ops.segment_sum(h[src2] * alpha[:, None], dst2, num_segments=n)
    return out + bias


def kernel(x, edge_index, edge_attr, batch_num_nodes, W_in, lin1, asrc1, adst1, line1, aedge1, b1, lin2, asrc2, adst2, line2, aedge2, b2, Wd1, bd1, gamma, beta, Wd2, bd2):
    n = x.shape[0]
    src, dst = edge_index[0], edge_index[1]
    node_in = x @ W_in
    h = jax.nn.relu(_gat_conv(node_in, src, dst, edge_attr, lin1, asrc1, adst1, line1, aedge1, b1, n))
    hs = _gat_conv(h, src, dst, edge_attr, lin2, asrc2, adst2, line2, aedge2, b2, n)
    poolidx = jnp.cumsum(batch_num_nodes) - 1
    onehot = jax.nn.one_hot(poolidx, n, dtype=hs.dtype)
    xs = onehot @ hs
    y = jax.nn.relu(xs @ Wd1 + bd1)
    y = _layernorm(y, gamma, beta)
    y = jax.nn.sigmoid(y @ Wd2 + bd2)
    return y


# speedup vs baseline: 2.6533x; 2.6533x over previous
import jax
import jax.numpy as jnp
from jax import lax
from jax.experimental import pallas as pl
from jax.experimental.pallas import tpu as pltpu
from jax.experimental.pallas import tpu_sc as plsc

NS = 16
LANES = 16
CH = 128
_NEG = -1e30


def _make_edge_sweep(nr, eprows, n):
    rw = eprows // NS

    def body(src_hbm, dst_hbm, ale_hbm, idx2_hbm, als_hbm, ald_hbm, m_hbm,
             h_hbm, zn_hbm, zd_hbm, nout, dout,
             als_v, ald_v, m_v, src_v, dst_v, ale_v, p_v, idx2_v,
             rows_v, acc_n, acc_d, gsem):
        sid = lax.axis_index("s")
        base = sid * rw

        pltpu.sync_copy(als_hbm, als_v)
        pltpu.sync_copy(ald_hbm, ald_v)
        pltpu.sync_copy(m_hbm, m_v)
        pltpu.sync_copy(zn_hbm, acc_n)
        pltpu.sync_copy(zd_hbm, acc_d)

        mv = m_v[...]
        lane = lax.iota(jnp.int32, LANES)

        def chunk_body(c, carry):
            pltpu.sync_copy(src_hbm.at[base + c], src_v)
            pltpu.sync_copy(dst_hbm.at[base + c], dst_v)
            pltpu.sync_copy(ale_hbm.at[base + c], ale_v)
            pltpu.sync_copy(idx2_hbm.at[base + c], idx2_v)
            desc = pltpu.async_copy(h_hbm.at[src_v], rows_v, gsem)

            for g in range(CH // LANES):
                sl = pl.ds(g * LANES, LANES)
                a = plsc.load_gather(als_v, [src_v[sl]])
                d = plsc.load_gather(ald_v, [dst_v[sl]])
                z = a + d + ale_v[sl]
                l = jnp.where(z >= 0, z, z * jnp.float32(0.2))
                p = jnp.exp(l - mv)
                p_v[sl] = p
                tg = idx2_v[sl]
                plsc.addupdate_scatter(acc_d, [tg * LANES + lane], p)

            desc.wait()

            def scale_row(r, _):
                ridx = jnp.full((LANES,), r, jnp.int32)
                pr = plsc.load_gather(p_v, [ridx])
                tgs = plsc.load_gather(idx2_v, [ridx]) * 128
                for f in range(128 // LANES):
                    fs = pl.ds(f * LANES, LANES)
                    v = rows_v[r, fs] * pr
                    plsc.addupdate_scatter(acc_n, [tgs + f * LANES + lane],
                                           v)
                return 0

            lax.fori_loop(0, CH, scale_row, 0)
            return carry

        lax.fori_loop(0, rw, chunk_body, 0)

        plsc.subcore_barrier()
        pltpu.sync_copy(acc_n, nout.at[sid])
        pltpu.sync_copy(acc_d, dout.at[sid])

    mesh = plsc.VectorSubcoreMesh(core_axis_name="c", subcore_axis_name="s",
                                  num_cores=1, num_subcores=NS)
    return pl.kernel(
        body,
        out_type=[jax.ShapeDtypeStruct((NS, nr * 128), jnp.float32),
                  jax.ShapeDtypeStruct((NS, nr * 16), jnp.float32)],
        mesh=mesh,
        compiler_params=pltpu.CompilerParams(needs_layout_passes=False),
        scratch_types=[
            pltpu.VMEM((n,), jnp.float32),
            pltpu.VMEM((n,), jnp.float32),
            pltpu.VMEM((16,), jnp.float32),
            pltpu.VMEM((CH,), jnp.int32),
            pltpu.VMEM((CH,), jnp.int32),
            pltpu.VMEM((CH,), jnp.float32),
            pltpu.VMEM((CH,), jnp.float32),
            pltpu.VMEM((CH,), jnp.int32),
            pltpu.VMEM((CH, 128), jnp.float32),
            pltpu.VMEM((nr * 128,), jnp.float32),
            pltpu.VMEM((nr * 16,), jnp.float32),
            pltpu.SemaphoreType.DMA,
        ],
        name=f"gat_edge_sweep_nr{nr}",
    )


def _stage_a(x, w_in, lin1, asrc1, adst1, nb, blk):
    def body(x_ref, wi_ref, l1_ref, as_ref, ad_ref,
             h1_ref, als_ref, ald_ref, amx_ref, bmx_ref):
        i = pl.program_id(0)
        h1 = (x_ref[...] @ wi_ref[...]) @ l1_ref[...]
        h1_ref[...] = h1
        als = h1 @ as_ref[...]
        ald = h1 @ ad_ref[...]
        als_ref[...] = als
        ald_ref[...] = ald

        @pl.when(i == 0)
        def _():
            amx_ref[...] = jnp.full((1, 1), _NEG, jnp.float32)
            bmx_ref[...] = jnp.full((1, 1), _NEG, jnp.float32)

        amx_ref[...] = jnp.maximum(amx_ref[...], jnp.max(als))
        bmx_ref[...] = jnp.maximum(bmx_ref[...], jnp.max(ald))

    n = x.shape[0]
    return pl.pallas_call(
        body,
        grid=(nb,),
        in_specs=[
            pl.BlockSpec((blk, 128), lambda i: (i, 0)),
            pl.BlockSpec((128, 128), lambda i: (0, 0)),
            pl.BlockSpec((128, 128), lambda i: (0, 0)),
            pl.BlockSpec((128, 1), lambda i: (0, 0)),
            pl.BlockSpec((128, 1), lambda i: (0, 0)),
        ],
        out_specs=[
            pl.BlockSpec((blk, 128), lambda i: (i, 0)),
            pl.BlockSpec((blk, 1), lambda i: (i, 0)),
            pl.BlockSpec((blk, 1), lambda i: (i, 0)),
            pl.BlockSpec((1, 1), lambda i: (0, 0)),
            pl.BlockSpec((1, 1), lambda i: (0, 0)),
        ],
        out_shape=[
            jax.ShapeDtypeStruct((n, 128), jnp.float32),
            jax.ShapeDtypeStruct((n, 1), jnp.float32),
            jax.ShapeDtypeStruct((n, 1), jnp.float32),
            jax.ShapeDtypeStruct((1, 1), jnp.float32),
            jax.ShapeDtypeStruct((1, 1), jnp.float32),
        ],
    )(x, w_in, lin1, asrc1, adst1)


def _stage_edges(ea2d, wsel1, wsel2, nb, blk):
    def body(ea_ref, w1_ref, w2_ref, a1_ref, a2_ref, s_ref, m1_ref, m2_ref):
        i = pl.program_id(0)
        xb = ea_ref[...]
        y1 = xb @ w1_ref[...]
        y2 = xb @ w2_ref[...]
        a1_ref[...] = y1
        a2_ref[...] = y2

        @pl.when(i == 0)
        def _():
            s_ref[...] = jnp.zeros((1, 128), jnp.float32)
            m1_ref[...] = jnp.full((1, 32), _NEG, jnp.float32)
            m2_ref[...] = jnp.full((1, 32), _NEG, jnp.float32)

        s_ref[...] = s_ref[...] + jnp.sum(xb, axis=0, keepdims=True)
        m1_ref[...] = jnp.maximum(m1_ref[...],
                                  jnp.max(y1, axis=0, keepdims=True))
        m2_ref[...] = jnp.maximum(m2_ref[...],
                                  jnp.max(y2, axis=0, keepdims=True))

    er = ea2d.shape[0]
    return pl.pallas_call(
        body,
        grid=(nb,),
        in_specs=[
            pl.BlockSpec((blk, 128), lambda i: (i, 0)),
            pl.BlockSpec((128, 32), lambda i: (0, 0)),
            pl.BlockSpec((128, 32), lambda i: (0, 0)),
        ],
        out_specs=[
            pl.BlockSpec((blk, 32), lambda i: (i, 0)),
            pl.BlockSpec((blk, 32), lambda i: (i, 0)),
            pl.BlockSpec((1, 128), lambda i: (0, 0)),
            pl.BlockSpec((1, 32), lambda i: (0, 0)),
            pl.BlockSpec((1, 32), lambda i: (0, 0)),
        ],
        out_shape=[
            jax.ShapeDtypeStruct((er, 32), jnp.float32),
            jax.ShapeDtypeStruct((er, 32), jnp.float32),
            jax.ShapeDtypeStruct((1, 128), jnp.float32),
            jax.ShapeDtypeStruct((1, 32), jnp.float32),
            jax.ShapeDtypeStruct((1, 32), jnp.float32),
        ],
    )(ea2d, wsel1, wsel2)


def _stage_c(na, dsum, h1, als1, ald1, m1, c1, b1, lin2, asrc2, adst2,
             ngrid, blk):
    def body(na_ref, ds_ref, h1_ref, as1_ref, ad1_ref, m1_ref,
             c1_ref, b1_ref, l2_ref, as2_ref, ad2_ref,
             h2_ref, als2_ref, ald2_ref, amx_ref, bmx_ref):
        i = pl.program_id(0)
        initl = as1_ref[...] + ad1_ref[...] + c1_ref[0, 0]
        initl = jnp.where(initl >= 0, initl, initl * jnp.float32(0.2))
        ini_d = jnp.exp(initl - m1_ref[0, 0])
        h1 = h1_ref[...]
        numer = na_ref[...] + ini_d * h1
        den = ds_ref[...] + ini_d + jnp.float32(1e-16)
        out1 = numer / den
        hr = jnp.maximum(out1 + b1_ref[...], 0.0)
        h2 = hr @ l2_ref[...]
        h2_ref[...] = h2
        als2 = h2 @ as2_ref[...]
        ald2 = h2 @ ad2_ref[...]
        als2_ref[...] = als2
        ald2_ref[...] = ald2

        @pl.when(i == 0)
        def _():
            amx_ref[...] = jnp.full((1, 1), _NEG, jnp.float32)
            bmx_ref[...] = jnp.full((1, 1), _NEG, jnp.float32)

        amx_ref[...] = jnp.maximum(amx_ref[...], jnp.max(als2))
        bmx_ref[...] = jnp.maximum(bmx_ref[...], jnp.max(ald2))

    n = h1.shape[0]
    return pl.pallas_call(
        body,
        grid=(ngrid,),
        in_specs=[
            pl.BlockSpec((blk, 128), lambda i: (i, 0)),
            pl.BlockSpec((blk, 1), lambda i: (i, 0)),
            pl.BlockSpec((blk, 128), lambda i: (i, 0)),
            pl.BlockSpec((blk, 1), lambda i: (i, 0)),
            pl.BlockSpec((blk, 1), lambda i: (i, 0)),
            pl.BlockSpec((1, 1), lambda i: (0, 0)),
            pl.BlockSpec((1, 1), lambda i: (0, 0)),
            pl.BlockSpec((1, 128), lambda i: (0, 0)),
            pl.BlockSpec((128, 128), lambda i: (0, 0)),
            pl.BlockSpec((128, 1), lambda i: (0, 0)),
            pl.BlockSpec((128, 1), lambda i: (0, 0)),
        ],
        out_specs=[
            pl.BlockSpec((blk, 128), lambda i: (i, 0)),
            pl.BlockSpec((blk, 1), lambda i: (i, 0)),
            pl.BlockSpec((blk, 1), lambda i: (i, 0)),
            pl.BlockSpec((1, 1), lambda i: (0, 0)),
            pl.BlockSpec((1, 1), lambda i: (0, 0)),
        ],
        out_shape=[
            jax.ShapeDtypeStruct((n, 128), jnp.float32),
            jax.ShapeDtypeStruct((n, 1), jnp.float32),
            jax.ShapeDtypeStruct((n, 1), jnp.float32),
            jax.ShapeDtypeStruct((1, 1), jnp.float32),
            jax.ShapeDtypeStruct((1, 1), jnp.float32),
        ],
    )(na, dsum, h1, als1, ald1, m1, c1, b1, lin2, asrc2, adst2)


def _stage_e(n2, oh, h2, als2, ald2, d2, m2, c2, b2, wd1, bd1, gamma, beta,
             wd2p, bd2p):
    def body(n2_ref, oh_ref, h2_ref, als2_ref, ald2_ref, d2_ref, m2_ref,
             c2_ref, b2_ref, w1_ref, b1_ref, g_ref, be_ref, w2_ref, bb_ref,
             o_ref):
        oh = oh_ref[...]
        h2p = oh @ h2_ref[...]
        als2p = oh @ als2_ref[...]
        ald2p = oh @ ald2_ref[...]
        initl = als2p + ald2p + c2_ref[0, 0]
        initl = jnp.where(initl >= 0, initl, initl * jnp.float32(0.2))
        selfd = jnp.exp(initl - m2_ref[0, 0])
        xs = (n2_ref[...] + selfd * h2p) / \
            (d2_ref[...] + selfd + jnp.float32(1e-16)) + b2_ref[...]
        y = jnp.maximum(xs @ w1_ref[...] + b1_ref[...], 0.0)
        mu = jnp.mean(y, axis=-1, keepdims=True)
        var = jnp.mean((y - mu) * (y - mu), axis=-1, keepdims=True)
        z = (y - mu) / jnp.sqrt(var + jnp.float32(1e-5)) * g_ref[...] \
            + be_ref[...]
        o = jax.nn.sigmoid(z @ w2_ref[...] + bb_ref[...])
        o_ref[...] = o

    return pl.pallas_call(
        body,
        out_shape=jax.ShapeDtypeStruct((16, 128), jnp.float32),
    )(n2, oh, h2, als2, ald2, d2, m2, c2, b2, wd1, bd1, gamma, beta,
      wd2p, bd2p)


def _lrelu(v):
    return jnp.where(v >= 0, v, v * jnp.float32(0.2))


def kernel(x, edge_index, edge_attr, batch_num_nodes, W_in, lin1, asrc1,
           adst1, line1, aedge1, b1, lin2, asrc2, adst2, line2, aedge2, b2,
           Wd1, bd1, gamma, beta, Wd2, bd2):
    n, _ = x.shape
    e = edge_index.shape[1]
    de = edge_attr.shape[1]
    nbatch = batch_num_nodes.shape[0]
    nt = Wd2.shape[1]
    f32 = jnp.float32

    perrow = 128 // de
    ea2d = edge_attr.reshape(e // perrow, 128)
    ep = ((e + NS * CH - 1) // (NS * CH)) * (NS * CH)
    eprows = ep // CH
    src_p = jnp.pad(edge_index[0], (0, ep - e)).reshape(eprows, CH)
    dst_p = jnp.pad(edge_index[1], (0, ep - e)).reshape(eprows, CH)

    we1 = line1 @ aedge1
    we2 = line2 @ aedge2
    ii = jnp.arange(128)
    sel = (ii[:, None] // de == jnp.arange(perrow)[None, :]).astype(f32)
    wsel1 = sel * we1[jnp.arange(128) % de][:, None]
    wsel2 = sel * we2[jnp.arange(128) % de][:, None]

    blk, nb = 400, n // 400
    h1, als1, ald1, amx1, bmx1 = _stage_a(
        x, W_in, lin1, asrc1.reshape(128, 1), adst1.reshape(128, 1), nb, blk)

    eb = ea2d.shape[0] // 400
    ale1p, ale2p, easum, alemx1, alemx2 = _stage_edges(
        ea2d, wsel1, wsel2, eb, 400)

    mean_ea = easum.reshape(perrow, de).sum(0) / f32(e)
    c1 = (mean_ea @ we1).reshape(1, 1)
    c2 = (mean_ea @ we2).reshape(1, 1)
    m1 = _lrelu(amx1[0, 0] + bmx1[0, 0]
                + jnp.maximum(jnp.max(alemx1), c1[0, 0])).reshape(1, 1)

    ale1 = jnp.pad(ale1p.reshape(e), (0, ep - e),
                   constant_values=_NEG).reshape(eprows, CH)
    ale2 = jnp.pad(ale2p.reshape(e), (0, ep - e),
                   constant_values=_NEG).reshape(eprows, CH)

    alsf = als1.reshape(n)
    aldf = ald1.reshape(n)
    srcf = edge_index[0]
    dstf = edge_index[1]
    alef = ale1.reshape(eprows * CH)[:e]
    pe = jnp.exp(_lrelu(alsf[srcf] + aldf[dstf] + alef) - m1[0, 0])
    numer1 = jax.ops.segment_sum(h1[srcf] * pe[:, None], dstf,
                                 num_segments=n)
    dsum1 = jax.ops.segment_sum(pe, dstf, num_segments=n).reshape(n, 1)

    h2, als2, ald2, amx2, bmx2 = _stage_c(
        numer1, dsum1, h1, als1, ald1, m1, c1,
        b1.reshape(1, 128), lin2, asrc2.reshape(128, 1),
        adst2.reshape(128, 1), nb, blk)

    m2 = _lrelu(amx2[0, 0] + bmx2[0, 0]
                + jnp.maximum(jnp.max(alemx2), c2[0, 0])).reshape(1, 1)
    m2vec = jnp.broadcast_to(m2.reshape(1), (16,))

    poolidx = jnp.cumsum(batch_num_nodes) - 1
    match = (jnp.arange(n, dtype=jnp.int32)[None, :]
             == poolidx[:, None].astype(jnp.int32))
    eqp = (dst_p[None, :, :] == poolidx[:, None, None].astype(jnp.int32))
    pid = (eqp * (jnp.arange(nbatch, dtype=jnp.int32)[:, None, None]
                  + 1)).sum(0)
    idx_p = jnp.where(pid > 0, pid - 1, nbatch).astype(jnp.int32)

    zn = jnp.zeros((32 * 128,), f32)
    zd = jnp.zeros((32 * 16,), f32)
    sweep2 = _make_edge_sweep(32, eprows, n)
    n2out, d2out = sweep2(src_p, dst_p, ale2, idx_p, als2.reshape(n),
                          ald2.reshape(n), m2vec, h2, zn, zd)
    n2 = n2out.reshape(NS, 32, 128).sum(0)[:16, :]
    d2 = d2out.reshape(NS, 32, 16).sum(0).sum(-1)[:16].reshape(16, 1)

    oh = match.astype(f32)
    wd2p = jnp.pad(Wd2, ((0, 0), (0, 128 - nt)))
    bd2p = jnp.pad(bd2, (0, 128 - nt)).reshape(1, 128)
    o = _stage_e(n2, oh, h2, als2, ald2, d2, m2, c2, b2.reshape(1, 128),
                 Wd1, bd1.reshape(1, 128), gamma.reshape(1, 128),
                 beta.reshape(1, 128), wd2p, bd2p)
    return o[:, :nt]
